# baseline (speedup 1.0000x reference)
import functools

import jax
import jax.numpy as jnp
from jax import lax
from jax.experimental import pallas as pl
from jax.experimental.pallas import tpu as pltpu
from jax.experimental.pallas import tpu_sc as plsc

N_SAMPLES = 3200000
N_RAYS = 100000
NC = 2
NS = 16
NW = NC * NS
RPW = 3128
ACC = RPW * 3
ACC_PAD = 9392
CH = 8192
GRP = CH // 16
SEARCH_ROUNDS = 10


def _sc_body(
    w_hbm, x_hbm, y_hbm, z_hbm, i_hbm, out_hbm,
    w_va, x_va, y_va, z_va, i_va,
    w_vb, x_vb, y_vb, z_vb, i_vb,
    acc_v, probe_v, sem_a, sem_b, sem_p,
):
    wid = lax.axis_index("s") * NC + lax.axis_index("c")
    ray_lo = wid * RPW
    rpw_w = jnp.minimum(RPW, N_RAYS - ray_lo)

    lane = lax.iota(jnp.int32, 16)
    sh_dn = jnp.maximum(lane - 1, 0)
    sh_up = jnp.minimum(lane + 1, 15)
    lane0 = lane == 0
    lane15 = lane == 15
    lhalf = lane < 8
    rhalf = lane >= 8

    tv = jnp.where(lhalf, ray_lo, ray_lo + rpw_w)

    def search_round(_, st):
        lo1, hi1, lo2, hi2 = st
        st1 = (hi1 - lo1 + 7) // 8
        st2 = (hi2 - lo2 + 7) // 8
        q = jnp.where(lhalf, lo1 + lane * st1, lo2 + (lane - 8) * st2)
        pc = jnp.minimum(q, N_SAMPLES - 1)
        pltpu.async_copy(i_hbm.at[pc], probe_v, sem_p).wait()
        pv = probe_v[pl.ds(0, 16)]
        bel = (pv < tv) & (q < N_SAMPLES)
        c1 = plsc.all_reduce_population_count(bel & lhalf)[0]
        c2 = plsc.all_reduce_population_count(bel & rhalf)[0]
        lo1n = jnp.where(c1 == 0, lo1, lo1 + (c1 - 1) * st1 + 1)
        hi1n = jnp.where(c1 >= 8, hi1, jnp.minimum(hi1, lo1 + c1 * st1))
        lo2n = jnp.where(c2 == 0, lo2, lo2 + (c2 - 1) * st2 + 1)
        hi2n = jnp.where(c2 >= 8, hi2, jnp.minimum(hi2, lo2 + c2 * st2))
        return lo1n, hi1n, lo2n, hi2n

    z = jnp.int32(0)
    n = jnp.int32(N_SAMPLES)
    start, _, end, _ = lax.fori_loop(
        0, SEARCH_ROUNDS, search_round, (z, n, z, n)
    )

    zeros16 = jnp.zeros((16,), jnp.float32)

    @plsc.parallel_loop(0, ACC_PAD // 16)
    def _(k):
        acc_v[pl.ds(k * 16, 16)] = zeros16

    base0 = start & (-8)
    nchunks = (end - base0 + CH - 1) // CH

    hbms = (w_hbm, x_hbm, y_hbm, z_hbm, i_hbm)
    set_a = (w_va, x_va, y_va, z_va, i_va)
    set_b = (w_vb, x_vb, y_vb, z_vb, i_vb)

    def chunk_base(ci):
        ub = base0 + ci * CH
        b = pl.multiple_of(jnp.minimum(ub, N_SAMPLES - CH), 8)
        return ub, b

    def start_dma(ci, bufs, sem):
        _, b = chunk_base(ci)
        for h, v in zip(hbms, bufs):
            pltpu.async_copy(h.at[pl.ds(b, CH)], v, sem)

    def wait_dma(bufs, sem):
        for h, v in zip(hbms, bufs):
            pltpu.make_async_copy(h.at[pl.ds(0, CH)], v, sem).wait()

    def compute(ci, bufs):
        w_v, x_v, y_v, z_v, i_v = bufs
        ub, b = chunk_base(ci)
        d = ub - b

        @plsc.parallel_loop(0, GRP, unroll=8)
        def _(j):
            j16 = j * 16
            idx16 = i_v[pl.ds(j16, 16)]
            w16 = w_v[pl.ds(j16, 16)]
            pos_ok = (j16 + lane) >= d
            w16z = jnp.where(pos_ok, w16, 0.0)
            lv = idx16 - ray_lo
            ray_ok = (lv >= 0) & (lv < rpw_w)
            tgt0 = jnp.clip(lv, 0, RPW - 1) * 3
            prev_idx = idx16.at[sh_dn].get(mode="promise_in_bounds")
            is_start = (idx16 != prev_idx) | lane0
            sp = plsc.cummax(jnp.where(is_start, lane, 0))
            endv = jnp.where(is_start, 1, 0).at[sh_up].get(
                mode="promise_in_bounds"
            )
            seg_end = (endv == 1) | lane15
            base_i = jnp.maximum(sp - 1, 0)
            has_prev = sp > 0
            m = seg_end & ray_ok
            for ch, c_v in enumerate((x_v, y_v, z_v)):
                oc = c_v[pl.ds(j16, 16)]
                cs = plsc.cumsum(w16z * oc)
                pb = cs.at[base_i].get(mode="promise_in_bounds")
                run = cs - jnp.where(has_prev, pb, 0.0)
                plsc.addupdate_scatter(acc_v, [tgt0 + ch], run, mask=m)

    pl.when(nchunks > 0)(lambda: start_dma(0, set_a, sem_a))
    npairs = (nchunks + 1) // 2

    def pair_body(cp, carry):
        ci0 = 2 * cp
        ci1 = ci0 + 1
        pl.when(ci1 < nchunks)(lambda: start_dma(ci1, set_b, sem_b))
        wait_dma(set_a, sem_a)
        compute(ci0, set_a)

        def second():
            pl.when(ci1 + 1 < nchunks)(
                lambda: start_dma(ci1 + 1, set_a, sem_a)
            )
            wait_dma(set_b, sem_b)
            compute(ci1, set_b)

        pl.when(ci1 < nchunks)(second)
        return carry

    lax.fori_loop(0, npairs, pair_body, None)

    pltpu.sync_copy(
        acc_v.at[pl.ds(0, ACC)],
        out_hbm.at[pl.ds(pl.multiple_of(wid * ACC, 8), ACC)],
    )


@jax.jit
def _run(w_flat, ox, oy, oz, ray_indices):
    mesh = plsc.VectorSubcoreMesh(core_axis_name="c", subcore_axis_name="s")
    f32buf = pltpu.VMEM((CH,), jnp.float32)
    i32buf = pltpu.VMEM((CH,), jnp.int32)
    k = functools.partial(
        pl.kernel,
        mesh=mesh,
        out_type=jax.ShapeDtypeStruct((NW * ACC,), jnp.float32),
        scratch_types=[
            f32buf, f32buf, f32buf, f32buf, i32buf,
            f32buf, f32buf, f32buf, f32buf, i32buf,
            pltpu.VMEM((ACC_PAD,), jnp.float32),
            pltpu.VMEM((16,), jnp.int32),
            pltpu.SemaphoreType.DMA,
            pltpu.SemaphoreType.DMA,
            pltpu.SemaphoreType.DMA,
        ],
        compiler_params=pltpu.CompilerParams(needs_layout_passes=False),
    )(_sc_body)
    return k(w_flat, ox, oy, oz, ray_indices)


def kernel(weights, offsets, ray_indices, num_rays):
    w_flat = weights[:, 0]
    ox, oy, oz = offsets[:, 0], offsets[:, 1], offsets[:, 2]
    out = _run(w_flat, ox, oy, oz, ray_indices)
    return out[: N_RAYS * 3].reshape(N_RAYS, 3)


# speedup vs baseline: 1.3810x; 1.3810x over previous
import functools

import jax
import jax.numpy as jnp
from jax import lax
from jax.experimental import pallas as pl
from jax.experimental.pallas import tpu as pltpu
from jax.experimental.pallas import tpu_sc as plsc

N_SAMPLES = 3200000
N_RAYS = 100000
NC = 2
NS = 16
NW = NC * NS
RPW = 3128
ACC = RPW * 3
ACC_PAD = 9392
CH = 8192
GRP = CH // 16
SEARCH_ROUNDS = 10


def _sc_body(
    w_hbm, x_hbm, y_hbm, z_hbm, i_hbm, out_hbm,
    w_va, x_va, y_va, z_va, i_va,
    w_vb, x_vb, y_vb, z_vb, i_vb,
    acc_v, probe_v, sem_a, sem_b, sem_p,
):
    wid = lax.axis_index("s") * NC + lax.axis_index("c")
    ray_lo = wid * RPW
    rpw_w = jnp.minimum(RPW, N_RAYS - ray_lo)

    lane = lax.iota(jnp.int32, 16)
    sh_dn = jnp.maximum(lane - 1, 0)
    sh_up = jnp.minimum(lane + 1, 15)
    lane0 = lane == 0
    lane15 = lane == 15
    lhalf = lane < 8
    rhalf = lane >= 8

    tv = jnp.where(lhalf, ray_lo, ray_lo + rpw_w)

    def search_round(_, st):
        lo1, hi1, lo2, hi2 = st
        st1 = (hi1 - lo1 + 7) // 8
        st2 = (hi2 - lo2 + 7) // 8
        q = jnp.where(lhalf, lo1 + lane * st1, lo2 + (lane - 8) * st2)
        pc = jnp.minimum(q, N_SAMPLES - 1)
        pltpu.async_copy(i_hbm.at[pc], probe_v, sem_p).wait()
        pv = probe_v[pl.ds(0, 16)]
        bel = (pv < tv) & (q < N_SAMPLES)
        c1 = plsc.all_reduce_population_count(bel & lhalf)[0]
        c2 = plsc.all_reduce_population_count(bel & rhalf)[0]
        lo1n = jnp.where(c1 == 0, lo1, lo1 + (c1 - 1) * st1 + 1)
        hi1n = jnp.where(c1 >= 8, hi1, jnp.minimum(hi1, lo1 + c1 * st1))
        lo2n = jnp.where(c2 == 0, lo2, lo2 + (c2 - 1) * st2 + 1)
        hi2n = jnp.where(c2 >= 8, hi2, jnp.minimum(hi2, lo2 + c2 * st2))
        return lo1n, hi1n, lo2n, hi2n

    z = jnp.int32(0)
    n = jnp.int32(N_SAMPLES)
    start, _, end, _ = lax.fori_loop(
        0, SEARCH_ROUNDS, search_round, (z, n, z, n)
    )

    zeros16 = jnp.zeros((16,), jnp.float32)

    @plsc.parallel_loop(0, ACC_PAD // 16)
    def _(k):
        acc_v[pl.ds(k * 16, 16)] = zeros16

    base0 = start & (-8)
    nchunks = (end - base0 + CH - 1) // CH

    hbms = (w_hbm, x_hbm, y_hbm, z_hbm, i_hbm)
    set_a = (w_va, x_va, y_va, z_va, i_va)
    set_b = (w_vb, x_vb, y_vb, z_vb, i_vb)

    def chunk_base(ci):
        ub = base0 + ci * CH
        b = pl.multiple_of(jnp.minimum(ub, N_SAMPLES - CH), 8)
        return ub, b

    def start_dma(ci, bufs, sem):
        _, b = chunk_base(ci)
        for h, v in zip(hbms, bufs):
            pltpu.async_copy(h.at[pl.ds(b, CH)], v, sem)

    def wait_dma(bufs, sem):
        for h, v in zip(hbms, bufs):
            pltpu.make_async_copy(h.at[pl.ds(0, CH)], v, sem).wait()

    def compute(ci, bufs):
        w_v, x_v, y_v, z_v, i_v = bufs
        ub, b = chunk_base(ci)
        d = ub - b

        @plsc.parallel_loop(0, GRP, unroll=4)
        def _(j):
            j16 = j * 16
            idx16 = i_v[pl.ds(j16, 16)]
            w16 = w_v[pl.ds(j16, 16)]
            pos_ok = (j16 + lane) >= d
            w16z = jnp.where(pos_ok, w16, 0.0)
            lv = idx16 - ray_lo
            ray_ok = (lv >= 0) & (lv < rpw_w)
            tgt0 = jnp.clip(lv, 0, RPW - 1) * 3
            prev_idx = idx16.at[sh_dn].get(mode="promise_in_bounds")
            is_start = (idx16 != prev_idx) | lane0
            sp = plsc.cummax(jnp.where(is_start, lane, 0))
            endv = jnp.where(is_start, 1, 0).at[sh_up].get(
                mode="promise_in_bounds"
            )
            seg_end = (endv == 1) | lane15
            base_i = jnp.maximum(sp - 1, 0)
            has_prev = sp > 0
            m = seg_end & ray_ok
            for ch, c_v in enumerate((x_v, y_v, z_v)):
                oc = c_v[pl.ds(j16, 16)]
                cs = plsc.cumsum(w16z * oc)
                pb = cs.at[base_i].get(mode="promise_in_bounds")
                run = cs - jnp.where(has_prev, pb, 0.0)
                plsc.addupdate_scatter(acc_v, [tgt0 + ch], run, mask=m)

    pl.when(nchunks > 0)(lambda: start_dma(0, set_a, sem_a))
    npairs = (nchunks + 1) // 2

    def pair_body(cp, carry):
        ci0 = 2 * cp
        ci1 = ci0 + 1
        pl.when(ci1 < nchunks)(lambda: start_dma(ci1, set_b, sem_b))
        wait_dma(set_a, sem_a)
        compute(ci0, set_a)

        def second():
            pl.when(ci1 + 1 < nchunks)(
                lambda: start_dma(ci1 + 1, set_a, sem_a)
            )
            wait_dma(set_b, sem_b)
            compute(ci1, set_b)

        pl.when(ci1 < nchunks)(second)
        return carry

    lax.fori_loop(0, npairs, pair_body, None)

    pltpu.sync_copy(
        acc_v.at[pl.ds(0, ACC)],
        out_hbm.at[pl.ds(pl.multiple_of(wid * ACC, 8), ACC)],
    )


@jax.jit
def _run(w_flat, ox, oy, oz, ray_indices):
    mesh = plsc.VectorSubcoreMesh(core_axis_name="c", subcore_axis_name="s")
    f32buf = pltpu.VMEM((CH,), jnp.float32)
    i32buf = pltpu.VMEM((CH,), jnp.int32)
    k = functools.partial(
        pl.kernel,
        mesh=mesh,
        out_type=jax.ShapeDtypeStruct((NW * ACC,), jnp.float32),
        scratch_types=[
            f32buf, f32buf, f32buf, f32buf, i32buf,
            f32buf, f32buf, f32buf, f32buf, i32buf,
            pltpu.VMEM((ACC_PAD,), jnp.float32),
            pltpu.VMEM((16,), jnp.int32),
            pltpu.SemaphoreType.DMA,
            pltpu.SemaphoreType.DMA,
            pltpu.SemaphoreType.DMA,
        ],
        compiler_params=pltpu.CompilerParams(needs_layout_passes=False),
    )(_sc_body)
    return k(w_flat, ox, oy, oz, ray_indices)


def kernel(weights, offsets, ray_indices, num_rays):
    w_flat = weights[:, 0]
    ox, oy, oz = offsets[:, 0], offsets[:, 1], offsets[:, 2]
    out = _run(w_flat, ox, oy, oz, ray_indices)
    return out[: N_RAYS * 3].reshape(N_RAYS, 3)


# speedup vs baseline: 2.4072x; 1.7431x over previous
import functools

import jax
import jax.numpy as jnp
from jax import lax
from jax.experimental import pallas as pl
from jax.experimental.pallas import tpu as pltpu
from jax.experimental.pallas import tpu_sc as plsc

N_SAMPLES = 3200000
N_RAYS = 100000
NC = 2
NS = 16
NW = NC * NS
RPW = 3128
ACC = RPW * 3
ACC_PAD = 9392
CH = 8192
GRP = CH // 16
SEARCH_ROUNDS = 10


def _sc_body(
    w_hbm, x_hbm, y_hbm, z_hbm, i_hbm, out_hbm,
    w_va, x_va, y_va, z_va, i_va,
    w_vb, x_vb, y_vb, z_vb, i_vb,
    acc_v, probe_v, sem_a, sem_b, sem_p,
):
    wid = lax.axis_index("s") * NC + lax.axis_index("c")
    ray_lo = wid * RPW
    rpw_w = jnp.minimum(RPW, N_RAYS - ray_lo)

    lane = lax.iota(jnp.int32, 16)
    sh_dn = jnp.maximum(lane - 1, 0)
    sh_up = jnp.minimum(lane + 1, 15)
    lane0 = lane == 0
    lane15 = lane == 15
    lhalf = lane < 8
    rhalf = lane >= 8

    tv = jnp.where(lhalf, ray_lo, ray_lo + rpw_w)

    def search_round(_, st):
        lo1, hi1, lo2, hi2 = st
        st1 = (hi1 - lo1 + 7) // 8
        st2 = (hi2 - lo2 + 7) // 8
        q = jnp.where(lhalf, lo1 + lane * st1, lo2 + (lane - 8) * st2)
        pc = jnp.minimum(q, N_SAMPLES - 1)
        pltpu.async_copy(i_hbm.at[pc], probe_v, sem_p).wait()
        pv = probe_v[pl.ds(0, 16)]
        bel = (pv < tv) & (q < N_SAMPLES)
        c1 = plsc.all_reduce_population_count(bel & lhalf)[0]
        c2 = plsc.all_reduce_population_count(bel & rhalf)[0]
        lo1n = jnp.where(c1 == 0, lo1, lo1 + (c1 - 1) * st1 + 1)
        hi1n = jnp.where(c1 >= 8, hi1, jnp.minimum(hi1, lo1 + c1 * st1))
        lo2n = jnp.where(c2 == 0, lo2, lo2 + (c2 - 1) * st2 + 1)
        hi2n = jnp.where(c2 >= 8, hi2, jnp.minimum(hi2, lo2 + c2 * st2))
        return lo1n, hi1n, lo2n, hi2n

    z = jnp.int32(0)
    n = jnp.int32(N_SAMPLES)
    start, _, end, _ = lax.fori_loop(
        0, SEARCH_ROUNDS, search_round, (z, n, z, n)
    )

    zeros16 = jnp.zeros((16,), jnp.float32)

    @plsc.parallel_loop(0, ACC_PAD // 16)
    def _(k):
        acc_v[pl.ds(k * 16, 16)] = zeros16

    base0 = start & (-8)
    nchunks = (end - base0 + CH - 1) // CH

    hbms = (w_hbm, x_hbm, y_hbm, z_hbm, i_hbm)
    set_a = (w_va, x_va, y_va, z_va, i_va)
    set_b = (w_vb, x_vb, y_vb, z_vb, i_vb)

    def chunk_base(ci):
        ub = base0 + ci * CH
        b = pl.multiple_of(jnp.minimum(ub, N_SAMPLES - CH), 8)
        return ub, b

    def start_dma(ci, bufs, sem):
        _, b = chunk_base(ci)
        for h, v in zip(hbms, bufs):
            pltpu.async_copy(h.at[pl.ds(b, CH)], v, sem)

    def wait_dma(bufs, sem):
        for h, v in zip(hbms, bufs):
            pltpu.make_async_copy(h.at[pl.ds(0, CH)], v, sem).wait()

    def compute(ci, bufs):
        w_v, x_v, y_v, z_v, i_v = bufs
        ub, b = chunk_base(ci)
        d = ub - b

        @plsc.parallel_loop(0, GRP, unroll=4)
        def _(j):
            j16 = j * 16
            idx16 = i_v[pl.ds(j16, 16)]
            w16 = w_v[pl.ds(j16, 16)]
            pos_ok = (j16 + lane) >= d
            w16z = jnp.where(pos_ok, w16, 0.0)
            lv = idx16 - ray_lo
            ray_ok = (lv >= 0) & (lv < rpw_w)
            tgt0 = jnp.clip(lv, 0, RPW - 1)
            prev_idx = idx16.at[sh_dn].get(mode="promise_in_bounds")
            is_start = (idx16 != prev_idx) | lane0
            sp = plsc.cummax(jnp.where(is_start, lane, 0))
            endv = jnp.where(is_start, 1, 0).at[sh_up].get(
                mode="promise_in_bounds"
            )
            seg_end = (endv == 1) | lane15
            base_i = jnp.maximum(sp - 1, 0)
            has_prev = sp > 0
            m = seg_end & ray_ok
            for ch, c_v in enumerate((x_v, y_v, z_v)):
                oc = c_v[pl.ds(j16, 16)]
                cs = plsc.cumsum(w16z * oc)
                pb = cs.at[base_i].get(mode="promise_in_bounds")
                run = cs - jnp.where(has_prev, pb, 0.0)
                plsc.addupdate_scatter(
                    acc_v, [tgt0 + ch * RPW], run, mask=m
                )

    pl.when(nchunks > 0)(lambda: start_dma(0, set_a, sem_a))
    npairs = (nchunks + 1) // 2

    def pair_body(cp, carry):
        ci0 = 2 * cp
        ci1 = ci0 + 1
        pl.when(ci1 < nchunks)(lambda: start_dma(ci1, set_b, sem_b))
        wait_dma(set_a, sem_a)
        compute(ci0, set_a)

        def second():
            pl.when(ci1 + 1 < nchunks)(
                lambda: start_dma(ci1 + 1, set_a, sem_a)
            )
            wait_dma(set_b, sem_b)
            compute(ci1, set_b)

        pl.when(ci1 < nchunks)(second)
        return carry

    lax.fori_loop(0, npairs, pair_body, None)

    for ch in range(3):
        pltpu.sync_copy(
            acc_v.at[pl.ds(ch * RPW, RPW)],
            out_hbm.at[
                pl.ds(pl.multiple_of(ch * (NW * RPW) + wid * RPW, 8), RPW)
            ],
        )


@jax.jit
def _run(w_flat, ox, oy, oz, ray_indices):
    mesh = plsc.VectorSubcoreMesh(core_axis_name="c", subcore_axis_name="s")
    f32buf = pltpu.VMEM((CH,), jnp.float32)
    i32buf = pltpu.VMEM((CH,), jnp.int32)
    k = functools.partial(
        pl.kernel,
        mesh=mesh,
        out_type=jax.ShapeDtypeStruct((NW * ACC,), jnp.float32),
        scratch_types=[
            f32buf, f32buf, f32buf, f32buf, i32buf,
            f32buf, f32buf, f32buf, f32buf, i32buf,
            pltpu.VMEM((ACC_PAD,), jnp.float32),
            pltpu.VMEM((16,), jnp.int32),
            pltpu.SemaphoreType.DMA,
            pltpu.SemaphoreType.DMA,
            pltpu.SemaphoreType.DMA,
        ],
        compiler_params=pltpu.CompilerParams(needs_layout_passes=False),
    )(_sc_body)
    return k(w_flat, ox, oy, oz, ray_indices)


def kernel(weights, offsets, ray_indices, num_rays):
    w_flat = weights[:, 0]
    ox, oy, oz = offsets[:, 0], offsets[:, 1], offsets[:, 2]
    out = _run(w_flat, ox, oy, oz, ray_indices)
    out3 = out.reshape(3, NW * RPW)[:, :N_RAYS]
    return jnp.swapaxes(out3, 0, 1)
